# baseline (speedup 1.0000x reference)
import functools
import jax
import jax.numpy as jnp
from jax import lax
from jax.experimental import pallas as pl
from jax.experimental.pallas import tpu as pltpu, tpu_sc as plsc

BATCH = 16384
NUM_FIELDS = 26
EMBED_DIM = 16
B = BATCH * NUM_FIELDS
NC = 2
NS = 16
NW = NC * NS
BPW = B // NW
LANES = 16
CHUNK = 1664
NCHUNK = BPW // CHUNK


def _sc_gather(x_resh, table, off_tiled):
    mesh = plsc.VectorSubcoreMesh(core_axis_name="c", subcore_axis_name="s")

    @functools.partial(
        pl.kernel,
        out_type=jax.ShapeDtypeStruct((B, EMBED_DIM), jnp.float32),
        mesh=mesh,
        scratch_types=[
            pltpu.VMEM((CHUNK,), jnp.int32),
            pltpu.VMEM((CHUNK,), jnp.int32),
            pltpu.VMEM((CHUNK,), jnp.int32),
            pltpu.VMEM((CHUNK, EMBED_DIM), jnp.float32),
            pltpu.VMEM((CHUNK, EMBED_DIM), jnp.float32),
            pltpu.SemaphoreType.DMA,
            pltpu.SemaphoreType.DMA,
        ],
        compiler_params=pltpu.CompilerParams(use_tc_tiling_on_sc=False),
    )
    def k(x_hbm, table_hbm, off_hbm, out_hbm,
          idx_a, idx_b, off_v, rows_a, rows_b, sem_a, sem_b):
        wid = lax.axis_index("s") * NC + lax.axis_index("c")
        base = wid * BPW

        idxs = (idx_a, idx_b)
        rows = (rows_a, rows_b)
        sems = (sem_a, sem_b)

        pltpu.sync_copy(off_hbm, off_v)

        def load_add(k_i, slot):
            pltpu.sync_copy(x_hbm.at[wid, k_i], idxs[slot])

            def body(i, _):
                s = pl.ds(i * LANES, LANES)
                idxs[slot][s] = idxs[slot][s] + off_v[s]
                return 0

            lax.fori_loop(0, CHUNK // LANES, body, 0, unroll=4)

        def gather_start(slot):
            return pltpu.async_copy(
                table_hbm.at[idxs[slot]], rows[slot], sems[slot]
            )

        load_add(0, 0)
        pending = gather_start(0)

        for k_i in range(1, NCHUNK):
            slot = k_i % 2
            load_add(k_i, slot)
            nxt = gather_start(slot)
            pending.wait()
            pltpu.sync_copy(
                rows[1 - slot],
                out_hbm.at[pl.ds(base + (k_i - 1) * CHUNK, CHUNK)],
            )
            pending = nxt

        pending.wait()
        pltpu.sync_copy(
            rows[(NCHUNK - 1) % 2],
            out_hbm.at[pl.ds(base + (NCHUNK - 1) * CHUNK, CHUNK)],
        )

    return k(x_resh, table, off_tiled)


@jax.jit
def kernel(x, table, offset):
    x32 = x.astype(jnp.int32).reshape(NW, NCHUNK, CHUNK)
    off_tiled = jnp.tile(offset.astype(jnp.int32), CHUNK // NUM_FIELDS)
    out = _sc_gather(x32, table, off_tiled)
    return out.reshape(BATCH, NUM_FIELDS, EMBED_DIM)


# speedup vs baseline: 1.2636x; 1.2636x over previous
import functools
import jax
import jax.numpy as jnp
from jax import lax
from jax.experimental import pallas as pl
from jax.experimental.pallas import tpu as pltpu, tpu_sc as plsc

BATCH = 16384
NUM_FIELDS = 26
EMBED_DIM = 16
NC = 2
NS = 16
NW = NC * NS
BPW = BATCH // NW
LANES = 16
JBLKS = BPW // LANES


def _sc_gather(x_t, table, off_b):
    mesh = plsc.VectorSubcoreMesh(core_axis_name="c", subcore_axis_name="s")

    @functools.partial(
        pl.kernel,
        out_type=jax.ShapeDtypeStruct((NUM_FIELDS, EMBED_DIM, BATCH), jnp.float32),
        mesh=mesh,
        scratch_types=[
            pltpu.VMEM((BPW,), jnp.int32),
            pltpu.VMEM((BPW,), jnp.int32),
            pltpu.VMEM((NUM_FIELDS, EMBED_DIM), jnp.int32),
            pltpu.VMEM((BPW, EMBED_DIM), jnp.float32),
            pltpu.VMEM((BPW, EMBED_DIM), jnp.float32),
            pltpu.VMEM((EMBED_DIM, BPW), jnp.float32),
            pltpu.VMEM((EMBED_DIM, BPW), jnp.float32),
            pltpu.SemaphoreType.DMA,
            pltpu.SemaphoreType.DMA,
            pltpu.SemaphoreType.DMA,
            pltpu.SemaphoreType.DMA,
        ],
        compiler_params=pltpu.CompilerParams(
            use_tc_tiling_on_sc=False, needs_layout_passes=False
        ),
    )
    def k(x_hbm, table_hbm, off_hbm, out_hbm,
          idx_a, idx_b, off_v, rows_a, rows_b, tr_a, tr_b,
          gsem_a, gsem_b, wsem_a, wsem_b):
        wid = lax.axis_index("s") * NC + lax.axis_index("c")
        b0 = wid * BPW

        idxs = (idx_a, idx_b)
        rows = (rows_a, rows_b)
        trs = (tr_a, tr_b)
        gsems = (gsem_a, gsem_b)
        wsems = (wsem_a, wsem_b)

        pltpu.sync_copy(off_hbm, off_v)
        iota = lax.iota(jnp.int32, LANES)

        def load_add(f, slot):
            pltpu.sync_copy(x_hbm.at[f, pl.ds(b0, BPW)], idxs[slot])
            off_vec = off_v[f, :]

            def body(i, _):
                s = pl.ds(i * LANES, LANES)
                idxs[slot][s] = idxs[slot][s] + off_vec
                return 0

            lax.fori_loop(0, BPW // LANES, body, 0, unroll=4)

        def gather_start(slot):
            return pltpu.async_copy(
                table_hbm.at[idxs[slot]], rows[slot], gsems[slot]
            )

        def transpose(slot):
            def body(jb, _):
                row_ids = iota + jb * LANES
                for e in range(EMBED_DIM):
                    col_ids = jnp.full((LANES,), e, dtype=jnp.int32)
                    v = plsc.load_gather(rows[slot], [row_ids, col_ids])
                    trs[slot][e, pl.ds(jb * LANES, LANES)] = v
                return 0

            lax.fori_loop(0, JBLKS, body, 0)

        def write_start(f, slot):
            return pltpu.async_copy(
                trs[slot], out_hbm.at[f, :, pl.ds(b0, BPW)], wsems[slot]
            )

        load_add(0, 0)
        g_pending = gather_start(0)
        w_pending = [None, None]

        for f in range(1, NUM_FIELDS + 1):
            slot = f % 2
            prev = 1 - slot
            if f < NUM_FIELDS:
                load_add(f, slot)
                g_next = gather_start(slot)
            g_pending.wait()
            if w_pending[prev] is not None:
                w_pending[prev].wait()
            transpose(prev)
            w_pending[prev] = write_start(f - 1, prev)
            if f < NUM_FIELDS:
                g_pending = g_next

        for d in w_pending:
            if d is not None:
                d.wait()

    return k(x_t, table, off_b)


@jax.jit
def kernel(x, table, offset):
    x_t = x.astype(jnp.int32).T
    off_b = jnp.tile(offset.astype(jnp.int32)[:, None], (1, EMBED_DIM))
    out_t = _sc_gather(x_t, table, off_b)
    return jnp.transpose(out_t, (2, 0, 1))
